# Initial kernel scaffold; baseline (speedup 1.0000x reference)
import jax
import jax.numpy as jnp
from jax.experimental import pallas as pl


def kernel(edge_index, s, v, dir_ij, r_ij, d_ij, params):
    raise NotImplementedError("write your pallas kernel here")


# speedup vs baseline: 3.0393x; 3.0393x over previous
import functools

import jax
import jax.numpy as jnp
from jax import lax
from jax.experimental import pallas as pl
from jax.experimental.pallas import tpu as pltpu
from jax.experimental.pallas import tpu_sc as plsc

N = 10000
E = 160000
C = 128
H = 8
DH = 16
CUTOFF = 5.0

NC = 2
NS = 16
NW = NC * NS

GB = 200
G_PER_W = E // NW
SB = 200
S_PER_T = E // NS
ROWS_PER_T = 624
ROWS_REM = N - NS * ROWS_PER_T
ZROWS = 156

BN = 400
BE = 400

_f32 = jnp.float32


def _silu(x):
    return x * jax.nn.sigmoid(x)


def _node_dense_body(s_ref, v_ref, ln_g, ln_b, wq, bq, wk, bk, wv1, bv1,
                     wv2, bv2, wvq, wvk,
                     q_ref, k_ref, val_ref, vq_ref, vdot_ref):
    s = s_ref[...]
    mu = jnp.mean(s, axis=-1, keepdims=True)
    xc = s - mu
    var = jnp.mean(xc * xc, axis=-1, keepdims=True)
    x = xc * lax.rsqrt(var + 1e-5) * ln_g[...] + ln_b[...]
    q_ref[...] = jnp.dot(x, wq[...], preferred_element_type=_f32) + bq[...]
    k_ref[...] = jnp.dot(x, wk[...], preferred_element_type=_f32) + bk[...]
    hv = _silu(jnp.dot(x, wv1[...], preferred_element_type=_f32) + bv1[...])
    val_ref[...] = jnp.dot(hv, wv2[...], preferred_element_type=_f32) + bv2[...]
    v2 = v_ref[...]
    vdot = jnp.zeros_like(s)
    for i in range(3):
        sl = slice(i * C, (i + 1) * C)
        vq_i = jnp.dot(v2[:, sl], wvq[...], preferred_element_type=_f32)
        vk_i = jnp.dot(v2[:, sl], wvk[...], preferred_element_type=_f32)
        vq_ref[:, sl] = vq_i
        vdot = vdot + vq_i * vk_i
    vdot_ref[...] = vdot


def _node_dense(s, v2d, p):
    grid = (N // BN,)
    row = lambda i: (i, 0)
    cst = lambda i: (0, 0)
    out = pl.pallas_call(
        _node_dense_body,
        grid=grid,
        in_specs=[
            pl.BlockSpec((BN, C), row), pl.BlockSpec((BN, 3 * C), row),
            pl.BlockSpec((1, C), cst), pl.BlockSpec((1, C), cst),
            pl.BlockSpec((C, C), cst), pl.BlockSpec((1, C), cst),
            pl.BlockSpec((C, C), cst), pl.BlockSpec((1, C), cst),
            pl.BlockSpec((C, C), cst), pl.BlockSpec((1, C), cst),
            pl.BlockSpec((C, 3 * C), cst), pl.BlockSpec((1, 3 * C), cst),
            pl.BlockSpec((C, C), cst), pl.BlockSpec((C, C), cst),
        ],
        out_specs=[
            pl.BlockSpec((BN, C), row), pl.BlockSpec((BN, C), row),
            pl.BlockSpec((BN, 3 * C), row), pl.BlockSpec((BN, 3 * C), row),
            pl.BlockSpec((BN, C), row),
        ],
        out_shape=[
            jax.ShapeDtypeStruct((N, C), _f32),
            jax.ShapeDtypeStruct((N, C), _f32),
            jax.ShapeDtypeStruct((N, 3 * C), _f32),
            jax.ShapeDtypeStruct((N, 3 * C), _f32),
            jax.ShapeDtypeStruct((N, C), _f32),
        ],
    )(s, v2d,
      p['ln_g'].reshape(1, C), p['ln_b'].reshape(1, C),
      p['Wq'], p['bq'].reshape(1, C), p['Wk'], p['bk'].reshape(1, C),
      p['Wv1'], p['bv1'].reshape(1, C), p['Wv2'], p['bv2'].reshape(1, 3 * C),
      p['Wvq'], p['Wvk'])
    return out


def _sc_gather(q, k, val, v2d, src, dst):
    mesh = plsc.VectorSubcoreMesh(core_axis_name="c", subcore_axis_name="s")

    @functools.partial(
        pl.kernel,
        mesh=mesh,
        out_type=[
            jax.ShapeDtypeStruct((E, C), _f32),
            jax.ShapeDtypeStruct((E, C), _f32),
            jax.ShapeDtypeStruct((E, 3 * C), _f32),
            jax.ShapeDtypeStruct((E, 3 * C), _f32),
        ],
        scratch_types=[
            pltpu.VMEM((GB,), jnp.int32),
            pltpu.VMEM((GB, C), _f32),
            pltpu.VMEM((GB, 3 * C), _f32),
        ],
    )
    def gather_kernel(q_hbm, k_hbm, val_hbm, v_hbm, src_hbm, dst_hbm,
                      qd_hbm, ks_hbm, vals_hbm, vs_hbm, idx_v, b128, b384):
        wid = lax.axis_index("s") * NC + lax.axis_index("c")
        base = wid * G_PER_W

        @pl.loop(0, G_PER_W // GB)
        def _(j):
            b = base + j * GB
            pltpu.sync_copy(dst_hbm.at[pl.ds(b, GB)], idx_v)
            pltpu.sync_copy(q_hbm.at[idx_v], b128)
            pltpu.sync_copy(b128, qd_hbm.at[pl.ds(b, GB)])
            pltpu.sync_copy(src_hbm.at[pl.ds(b, GB)], idx_v)
            pltpu.sync_copy(k_hbm.at[idx_v], b128)
            pltpu.sync_copy(b128, ks_hbm.at[pl.ds(b, GB)])
            pltpu.sync_copy(val_hbm.at[idx_v], b384)
            pltpu.sync_copy(b384, vals_hbm.at[pl.ds(b, GB)])
            pltpu.sync_copy(v_hbm.at[idx_v], b384)
            pltpu.sync_copy(b384, vs_hbm.at[pl.ds(b, GB)])

    return gather_kernel(q, k, val, v2d, src, dst)


def _edge_body(r_ref, qd_ref, ks_ref, vals_ref, vs_ref, dir_ref, d_ref,
               wsig, bsig, wra, bra,
               m0_ref, m1_ref, m2_ref, m3_ref):
    r = r_ref[...]
    rk = _silu(jnp.dot(r, wsig[...], preferred_element_type=_f32) + bsig[...])
    ra = jnp.dot(r, wra[...], preferred_element_type=_f32) + bra[...]
    prod = qd_ref[...] * ks_ref[...] * rk
    attn = jnp.sum(prod.reshape(BE, H, DH), axis=-1)
    d = d_ref[...]
    cc = 0.5 * (jnp.cos(d * (jnp.pi / CUTOFF)) + 1.0) * (d < CUTOFF).astype(_f32)
    attn = _silu(attn) * cc
    attn128 = jnp.broadcast_to(attn[:, :, None], (BE, H, DH)).reshape(BE, C)
    val_j = vals_ref[...] * ra
    m0_ref[...] = val_j[:, :C] * attn128
    vec1 = val_j[:, C:2 * C]
    vec2 = val_j[:, 2 * C:]
    vs = vs_ref[...]
    dirs = dir_ref[...]
    m1_ref[...] = vs[:, :C] * vec1 + dirs[:, 0:1] * vec2
    m2_ref[...] = vs[:, C:2 * C] * vec1 + dirs[:, 1:2] * vec2
    m3_ref[...] = vs[:, 2 * C:] * vec1 + dirs[:, 2:3] * vec2


def _edge_stage(r_ij, qd, ks, vals, vs, dir_ij, d2, p):
    grid = (E // BE,)
    row = lambda i: (i, 0)
    cst = lambda i: (0, 0)
    return pl.pallas_call(
        _edge_body,
        grid=grid,
        in_specs=[
            pl.BlockSpec((BE, C), row), pl.BlockSpec((BE, C), row),
            pl.BlockSpec((BE, C), row), pl.BlockSpec((BE, 3 * C), row),
            pl.BlockSpec((BE, 3 * C), row), pl.BlockSpec((BE, 3), row),
            pl.BlockSpec((BE, 1), row),
            pl.BlockSpec((C, C), cst), pl.BlockSpec((1, C), cst),
            pl.BlockSpec((C, 3 * C), cst), pl.BlockSpec((1, 3 * C), cst),
        ],
        out_specs=[pl.BlockSpec((BE, C), row)] * 4,
        out_shape=[jax.ShapeDtypeStruct((E, C), _f32)] * 4,
    )(r_ij, qd, ks, vals, vs, dir_ij, d2,
      p['Wsig'], p['bsig'].reshape(1, C), p['Wra'], p['bra'].reshape(1, 3 * C))


def _sc_scatter(m0, m1, m2, m3, dst):
    mesh = plsc.VectorSubcoreMesh(core_axis_name="c", subcore_axis_name="s")

    @functools.partial(
        pl.kernel,
        mesh=mesh,
        out_type=[jax.ShapeDtypeStruct((N, C), _f32)] * 4,
        scratch_types=[
            pltpu.VMEM((SB,), jnp.int32),
            pltpu.VMEM((SB, C), _f32),
            pltpu.VMEM((ZROWS, C), _f32),
            pltpu.VMEM_SHARED((N, C), _f32),
        ],
    )
    def scatter_kernel(m0_hbm, m1_hbm, m2_hbm, m3_hbm, dst_hbm,
                       ds_hbm, dv0_hbm, dv1_hbm, dv2_hbm,
                       idx_v, mbuf, zbuf, acc):
        c = lax.axis_index("c")
        s = lax.axis_index("s")

        @pl.loop(0, ZROWS)
        def _(i):
            @pl.loop(0, C, step=16)
            def _(j):
                zbuf[i, pl.ds(j, 16)] = jnp.zeros((16,), _f32)

        def process(m_hbm, out_hbm):
            @pl.loop(0, ROWS_PER_T // ZROWS)
            def _(t):
                pltpu.sync_copy(zbuf, acc.at[pl.ds(s * ROWS_PER_T + t * ZROWS, ZROWS)])

            @pl.when(s == 0)
            def _():
                pltpu.sync_copy(zbuf.at[pl.ds(0, ROWS_REM)],
                                acc.at[pl.ds(NS * ROWS_PER_T, ROWS_REM)])
            plsc.subcore_barrier()
            @pl.loop(0, S_PER_T // SB)
            def _(j):
                b = s * S_PER_T + j * SB
                pltpu.sync_copy(dst_hbm.at[pl.ds(b, SB)], idx_v)
                pltpu.sync_copy(m_hbm.at[pl.ds(b, SB)], mbuf)
                pltpu.sync_copy(mbuf, acc.at[idx_v], add=True)
            plsc.subcore_barrier()
            pltpu.sync_copy(acc.at[pl.ds(s * ROWS_PER_T, ROWS_PER_T)],
                            out_hbm.at[pl.ds(s * ROWS_PER_T, ROWS_PER_T)])

            @pl.when(s == 0)
            def _():
                pltpu.sync_copy(acc.at[pl.ds(NS * ROWS_PER_T, ROWS_REM)],
                                out_hbm.at[pl.ds(NS * ROWS_PER_T, ROWS_REM)])
            plsc.subcore_barrier()

        @pl.when(c == 0)
        def _():
            process(m0_hbm, ds_hbm)
            process(m1_hbm, dv0_hbm)

        @pl.when(c == 1)
        def _():
            process(m2_hbm, dv1_hbm)
            process(m3_hbm, dv2_hbm)

    return scatter_kernel(m0, m1, m2, m3, dst)


def _post_body(ds_ref, dv0_ref, dv1_ref, dv2_ref, s_ref, v_ref, vq_ref,
               vdot_ref, ws1, bs1, ws2, bs2, sout_ref, vout_ref):
    hs = _silu(jnp.dot(ds_ref[...], ws1[...], preferred_element_type=_f32) + bs1[...])
    o = jnp.dot(hs, ws2[...], preferred_element_type=_f32) + bs2[...]
    o1 = o[:, :C]
    o2 = o[:, C:2 * C]
    o3 = o[:, 2 * C:]
    sout_ref[...] = s_ref[...] + o2 + o3 * vdot_ref[...]
    v2 = v_ref[...]
    vq = vq_ref[...]
    dvs = (dv0_ref, dv1_ref, dv2_ref)
    for i in range(3):
        sl = slice(i * C, (i + 1) * C)
        vout_ref[:, i, :] = v2[:, sl] + dvs[i][...] + o1 * vq[:, sl]


def _post_stage(ds, dv0, dv1, dv2, s, v2d, vq2d, vdot, p):
    grid = (N // BN,)
    row = lambda i: (i, 0)
    cst = lambda i: (0, 0)
    return pl.pallas_call(
        _post_body,
        grid=grid,
        in_specs=[
            pl.BlockSpec((BN, C), row), pl.BlockSpec((BN, C), row),
            pl.BlockSpec((BN, C), row), pl.BlockSpec((BN, C), row),
            pl.BlockSpec((BN, C), row), pl.BlockSpec((BN, 3 * C), row),
            pl.BlockSpec((BN, 3 * C), row), pl.BlockSpec((BN, C), row),
            pl.BlockSpec((C, C), cst), pl.BlockSpec((1, C), cst),
            pl.BlockSpec((C, 3 * C), cst), pl.BlockSpec((1, 3 * C), cst),
        ],
        out_specs=[
            pl.BlockSpec((BN, C), row),
            pl.BlockSpec((BN, 3, C), lambda i: (i, 0, 0)),
        ],
        out_shape=[
            jax.ShapeDtypeStruct((N, C), _f32),
            jax.ShapeDtypeStruct((N, 3, C), _f32),
        ],
    )(ds, dv0, dv1, dv2, s, v2d, vq2d, vdot,
      p['Ws1'], p['bs1'].reshape(1, C), p['Ws2'], p['bs2'].reshape(1, 3 * C))


@jax.jit
def _impl(edge_index, s, v, dir_ij, r_ij, d_ij, params):
    src = edge_index[0]
    dst = edge_index[1]
    v2d = v.reshape(N, 3 * C)
    d2 = d_ij.reshape(E, 1)

    q, k, val, vq2d, vdot = _node_dense(s, v2d, params)
    qd, ks, vals, vs = _sc_gather(q, k, val, v2d, src, dst)
    m0, m1, m2, m3 = _edge_stage(r_ij, qd, ks, vals, vs, dir_ij, d2, params)
    ds, dv0, dv1, dv2 = _sc_scatter(m0, m1, m2, m3, dst)
    s_out, v_out = _post_stage(ds, dv0, dv1, dv2, s, v2d, vq2d, vdot, params)
    return (s_out, v_out)


def kernel(edge_index, s, v, dir_ij, r_ij, d_ij, params):
    return _impl(edge_index, s, v, dir_ij, r_ij, d_ij, params)


# speedup vs baseline: 3.1019x; 1.0206x over previous
import functools

import jax
import jax.numpy as jnp
from jax import lax
from jax.experimental import pallas as pl
from jax.experimental.pallas import tpu as pltpu
from jax.experimental.pallas import tpu_sc as plsc

N = 10000
E = 160000
C = 128
H = 8
DH = 16
CUTOFF = 5.0

NC = 2
NS = 16
NW = NC * NS

GB1 = 200
GB2 = 40
G_PER_W = E // NW
SB = 80
S_PER_T = E // NS
ROWS_PER_T = 624
ROWS_REM = N - NS * ROWS_PER_T
ZROWS = 48

BN = 400
BE = 400

_f32 = jnp.float32


def _silu(x):
    return x * jax.nn.sigmoid(x)


def _node_dense_body(s_ref, v_ref, ln_g, ln_b, wq, bq, wk, bk, wv1, bv1,
                     wv2, bv2, wvq, wvk,
                     q_ref, k_ref, val_ref, vq_ref, vdot_ref):
    s = s_ref[...]
    mu = jnp.mean(s, axis=-1, keepdims=True)
    xc = s - mu
    var = jnp.mean(xc * xc, axis=-1, keepdims=True)
    x = xc * lax.rsqrt(var + 1e-5) * ln_g[...] + ln_b[...]
    q_ref[...] = jnp.dot(x, wq[...], preferred_element_type=_f32) + bq[...]
    k_ref[...] = jnp.dot(x, wk[...], preferred_element_type=_f32) + bk[...]
    hv = _silu(jnp.dot(x, wv1[...], preferred_element_type=_f32) + bv1[...])
    val_ref[...] = jnp.dot(hv, wv2[...], preferred_element_type=_f32) + bv2[...]
    v2 = v_ref[...]
    vdot = jnp.zeros_like(s)
    for i in range(3):
        sl = slice(i * C, (i + 1) * C)
        vq_i = jnp.dot(v2[:, sl], wvq[...], preferred_element_type=_f32)
        vk_i = jnp.dot(v2[:, sl], wvk[...], preferred_element_type=_f32)
        vq_ref[:, sl] = vq_i
        vdot = vdot + vq_i * vk_i
    vdot_ref[...] = vdot


def _node_dense(s, v2d, p):
    grid = (N // BN,)
    row = lambda i: (i, 0)
    cst = lambda i: (0, 0)
    out = pl.pallas_call(
        _node_dense_body,
        grid=grid,
        in_specs=[
            pl.BlockSpec((BN, C), row), pl.BlockSpec((BN, 3 * C), row),
            pl.BlockSpec((1, C), cst), pl.BlockSpec((1, C), cst),
            pl.BlockSpec((C, C), cst), pl.BlockSpec((1, C), cst),
            pl.BlockSpec((C, C), cst), pl.BlockSpec((1, C), cst),
            pl.BlockSpec((C, C), cst), pl.BlockSpec((1, C), cst),
            pl.BlockSpec((C, 3 * C), cst), pl.BlockSpec((1, 3 * C), cst),
            pl.BlockSpec((C, C), cst), pl.BlockSpec((C, C), cst),
        ],
        out_specs=[
            pl.BlockSpec((BN, C), row), pl.BlockSpec((BN, C), row),
            pl.BlockSpec((BN, 3 * C), row), pl.BlockSpec((BN, 3 * C), row),
            pl.BlockSpec((BN, C), row),
        ],
        out_shape=[
            jax.ShapeDtypeStruct((N, C), _f32),
            jax.ShapeDtypeStruct((N, C), _f32),
            jax.ShapeDtypeStruct((N, 3 * C), _f32),
            jax.ShapeDtypeStruct((N, 3 * C), _f32),
            jax.ShapeDtypeStruct((N, C), _f32),
        ],
    )(s, v2d,
      p['ln_g'].reshape(1, C), p['ln_b'].reshape(1, C),
      p['Wq'], p['bq'].reshape(1, C), p['Wk'], p['bk'].reshape(1, C),
      p['Wv1'], p['bv1'].reshape(1, C), p['Wv2'], p['bv2'].reshape(1, 3 * C),
      p['Wvq'], p['Wvk'])
    return out


def _sc_gather(q, k, val, v2d, src, dst):
    mesh = plsc.VectorSubcoreMesh(core_axis_name="c", subcore_axis_name="s")

    @functools.partial(
        pl.kernel,
        mesh=mesh,
        out_type=[
            jax.ShapeDtypeStruct((E, C), _f32),
            jax.ShapeDtypeStruct((E, C), _f32),
            jax.ShapeDtypeStruct((E, 3 * C), _f32),
            jax.ShapeDtypeStruct((E, 3 * C), _f32),
        ],
        scratch_types=[
            pltpu.VMEM((GB1,), jnp.int32),
            pltpu.VMEM((GB1, C), _f32),
            pltpu.VMEM((GB1, 3 * C), _f32),
        ],
    )
    def gather_kernel(q_hbm, k_hbm, val_hbm, v_hbm, src_hbm, dst_hbm,
                      qd_hbm, ks_hbm, vals_hbm, vs_hbm, idx_v, b128, b384):
        wid = lax.axis_index("s") * NC + lax.axis_index("c")
        base = wid * G_PER_W

        @pl.loop(0, G_PER_W // GB1)
        def _(j):
            b = base + j * GB1
            pltpu.sync_copy(dst_hbm.at[pl.ds(b, GB1)], idx_v)
            pltpu.sync_copy(q_hbm.at[idx_v], b128)
            pltpu.sync_copy(b128, qd_hbm.at[pl.ds(b, GB1)])
            pltpu.sync_copy(src_hbm.at[pl.ds(b, GB1)], idx_v)
            pltpu.sync_copy(k_hbm.at[idx_v], b128)
            pltpu.sync_copy(b128, ks_hbm.at[pl.ds(b, GB1)])
            pltpu.sync_copy(val_hbm.at[idx_v], b384)
            pltpu.sync_copy(b384, vals_hbm.at[pl.ds(b, GB1)])
            pltpu.sync_copy(v_hbm.at[idx_v], b384)
            pltpu.sync_copy(b384, vs_hbm.at[pl.ds(b, GB1)])

    return gather_kernel(q, k, val, v2d, src, dst)


def _edge_body(r_ref, qd_ref, ks_ref, vals_ref, vs_ref, dir_ref, d_ref,
               wsig, bsig, wra, bra,
               m0_ref, m1_ref, m2_ref, m3_ref):
    r = r_ref[...]
    rk = _silu(jnp.dot(r, wsig[...], preferred_element_type=_f32) + bsig[...])
    ra = jnp.dot(r, wra[...], preferred_element_type=_f32) + bra[...]
    prod = qd_ref[...] * ks_ref[...] * rk
    attn = jnp.sum(prod.reshape(BE, H, DH), axis=-1)
    d = d_ref[...]
    cc = 0.5 * (jnp.cos(d * (jnp.pi / CUTOFF)) + 1.0) * (d < CUTOFF).astype(_f32)
    attn = _silu(attn) * cc
    attn128 = jnp.broadcast_to(attn[:, :, None], (BE, H, DH)).reshape(BE, C)
    val_j = vals_ref[...] * ra
    m0_ref[...] = val_j[:, :C] * attn128
    vec1 = val_j[:, C:2 * C]
    vec2 = val_j[:, 2 * C:]
    vs = vs_ref[...]
    dirs = dir_ref[...]
    m1_ref[...] = vs[:, :C] * vec1 + dirs[:, 0:1] * vec2
    m2_ref[...] = vs[:, C:2 * C] * vec1 + dirs[:, 1:2] * vec2
    m3_ref[...] = vs[:, 2 * C:] * vec1 + dirs[:, 2:3] * vec2


def _edge_stage(r_ij, qd, ks, vals, vs, dir_ij, d2, p):
    grid = (E // BE,)
    row = lambda i: (i, 0)
    cst = lambda i: (0, 0)
    return pl.pallas_call(
        _edge_body,
        grid=grid,
        in_specs=[
            pl.BlockSpec((BE, C), row), pl.BlockSpec((BE, C), row),
            pl.BlockSpec((BE, C), row), pl.BlockSpec((BE, 3 * C), row),
            pl.BlockSpec((BE, 3 * C), row), pl.BlockSpec((BE, 3), row),
            pl.BlockSpec((BE, 1), row),
            pl.BlockSpec((C, C), cst), pl.BlockSpec((1, C), cst),
            pl.BlockSpec((C, 3 * C), cst), pl.BlockSpec((1, 3 * C), cst),
        ],
        out_specs=[pl.BlockSpec((BE, C), row)] * 4,
        out_shape=[jax.ShapeDtypeStruct((E, C), _f32)] * 4,
    )(r_ij, qd, ks, vals, vs, dir_ij, d2,
      p['Wsig'], p['bsig'].reshape(1, C), p['Wra'], p['bra'].reshape(1, 3 * C))


def _sc_scatter(m0, m1, m2, m3, dst):
    mesh = plsc.VectorSubcoreMesh(core_axis_name="c", subcore_axis_name="s")
    nb = S_PER_T // SB

    @functools.partial(
        pl.kernel,
        mesh=mesh,
        out_type=[jax.ShapeDtypeStruct((N, C), _f32)] * 4,
        scratch_types=[
            pltpu.VMEM((SB,), jnp.int32),
            pltpu.VMEM((SB,), jnp.int32),
            pltpu.VMEM((SB, C), _f32),
            pltpu.VMEM((SB, C), _f32),
            pltpu.VMEM((ZROWS, C), _f32),
            pltpu.VMEM_SHARED((N, C), _f32),
            pltpu.SemaphoreType.DMA,
            pltpu.SemaphoreType.DMA,
            pltpu.SemaphoreType.DMA,
            pltpu.SemaphoreType.DMA,
        ],
    )
    def scatter_kernel(m0_hbm, m1_hbm, m2_hbm, m3_hbm, dst_hbm,
                       ds_hbm, dv0_hbm, dv1_hbm, dv2_hbm,
                       ix0, ix1, mb0, mb1, zbuf, acc, sl0, sl1, sa0, sa1):
        c = lax.axis_index("c")
        s = lax.axis_index("s")

        @pl.loop(0, ZROWS)
        def _(i):
            @pl.loop(0, C, step=16)
            def _(j):
                zbuf[i, pl.ds(j, 16)] = jnp.zeros((16,), _f32)

        def process(m_hbm, out_hbm):
            @pl.loop(0, ROWS_PER_T // ZROWS)
            def _(t):
                pltpu.sync_copy(zbuf, acc.at[pl.ds(s * ROWS_PER_T + t * ZROWS, ZROWS)])

            @pl.when(s == 0)
            def _():
                pltpu.sync_copy(zbuf.at[pl.ds(0, ROWS_REM)],
                                acc.at[pl.ds(NS * ROWS_PER_T, ROWS_REM)])
            plsc.subcore_barrier()

            def load(j, buf, sem):
                pltpu.async_copy(m_hbm.at[pl.ds(s * S_PER_T + j * SB, SB)], buf, sem)

            def wait_l(j, buf, sem):
                pltpu.make_async_copy(m_hbm.at[pl.ds(s * S_PER_T + j * SB, SB)],
                                      buf, sem).wait()

            load(0, mb0, sl0)

            @pl.loop(0, nb // 2)
            def _(t):
                j0 = 2 * t
                j1 = j0 + 1
                pltpu.sync_copy(dst_hbm.at[pl.ds(s * S_PER_T + j0 * SB, SB)], ix0)
                wait_l(j0, mb0, sl0)
                load(j1, mb1, sl1)
                pltpu.sync_copy(mb0, acc.at[ix0], add=True)
                pltpu.sync_copy(dst_hbm.at[pl.ds(s * S_PER_T + j1 * SB, SB)], ix1)
                wait_l(j1, mb1, sl1)

                @pl.when(j0 + 2 < nb)
                def _():
                    load(j0 + 2, mb0, sl0)
                pltpu.sync_copy(mb1, acc.at[ix1], add=True)

            if nb % 2 == 1:
                jt = nb - 1
                pltpu.sync_copy(dst_hbm.at[pl.ds(s * S_PER_T + jt * SB, SB)], ix0)
                wait_l(jt, mb0, sl0)
                pltpu.sync_copy(mb0, acc.at[ix0], add=True)
            plsc.subcore_barrier()
            pltpu.sync_copy(acc.at[pl.ds(s * ROWS_PER_T, ROWS_PER_T)],
                            out_hbm.at[pl.ds(s * ROWS_PER_T, ROWS_PER_T)])

            @pl.when(s == 0)
            def _():
                pltpu.sync_copy(acc.at[pl.ds(NS * ROWS_PER_T, ROWS_REM)],
                                out_hbm.at[pl.ds(NS * ROWS_PER_T, ROWS_REM)])
            plsc.subcore_barrier()

        @pl.when(c == 0)
        def _():
            process(m0_hbm, ds_hbm)
            process(m1_hbm, dv0_hbm)

        @pl.when(c == 1)
        def _():
            process(m2_hbm, dv1_hbm)
            process(m3_hbm, dv2_hbm)

    return scatter_kernel(m0, m1, m2, m3, dst)


def _post_body(ds_ref, dv0_ref, dv1_ref, dv2_ref, s_ref, v_ref, vq_ref,
               vdot_ref, ws1, bs1, ws2, bs2, sout_ref, vout_ref):
    hs = _silu(jnp.dot(ds_ref[...], ws1[...], preferred_element_type=_f32) + bs1[...])
    o = jnp.dot(hs, ws2[...], preferred_element_type=_f32) + bs2[...]
    o1 = o[:, :C]
    o2 = o[:, C:2 * C]
    o3 = o[:, 2 * C:]
    sout_ref[...] = s_ref[...] + o2 + o3 * vdot_ref[...]
    v2 = v_ref[...]
    vq = vq_ref[...]
    dvs = (dv0_ref, dv1_ref, dv2_ref)
    for i in range(3):
        sl = slice(i * C, (i + 1) * C)
        vout_ref[:, i, :] = v2[:, sl] + dvs[i][...] + o1 * vq[:, sl]


def _post_stage(ds, dv0, dv1, dv2, s, v2d, vq2d, vdot, p):
    grid = (N // BN,)
    row = lambda i: (i, 0)
    cst = lambda i: (0, 0)
    return pl.pallas_call(
        _post_body,
        grid=grid,
        in_specs=[
            pl.BlockSpec((BN, C), row), pl.BlockSpec((BN, C), row),
            pl.BlockSpec((BN, C), row), pl.BlockSpec((BN, C), row),
            pl.BlockSpec((BN, C), row), pl.BlockSpec((BN, 3 * C), row),
            pl.BlockSpec((BN, 3 * C), row), pl.BlockSpec((BN, C), row),
            pl.BlockSpec((C, C), cst), pl.BlockSpec((1, C), cst),
            pl.BlockSpec((C, 3 * C), cst), pl.BlockSpec((1, 3 * C), cst),
        ],
        out_specs=[
            pl.BlockSpec((BN, C), row),
            pl.BlockSpec((BN, 3, C), lambda i: (i, 0, 0)),
        ],
        out_shape=[
            jax.ShapeDtypeStruct((N, C), _f32),
            jax.ShapeDtypeStruct((N, 3, C), _f32),
        ],
    )(ds, dv0, dv1, dv2, s, v2d, vq2d, vdot,
      p['Ws1'], p['bs1'].reshape(1, C), p['Ws2'], p['bs2'].reshape(1, 3 * C))


@jax.jit
def _impl(edge_index, s, v, dir_ij, r_ij, d_ij, params):
    src = edge_index[0]
    dst = edge_index[1]
    v2d = v.reshape(N, 3 * C)
    d2 = d_ij.reshape(E, 1)

    q, k, val, vq2d, vdot = _node_dense(s, v2d, params)
    qd, ks, vals, vs = _sc_gather(q, k, val, v2d, src, dst)
    m0, m1, m2, m3 = _edge_stage(r_ij, qd, ks, vals, vs, dir_ij, d2, params)
    ds, dv0, dv1, dv2 = _sc_scatter(m0, m1, m2, m3, dst)
    s_out, v_out = _post_stage(ds, dv0, dv1, dv2, s, v2d, vq2d, vdot, params)
    return (s_out, v_out)


def kernel(edge_index, s, v, dir_ij, r_ij, d_ij, params):
    return _impl(edge_index, s, v, dir_ij, r_ij, d_ij, params)


# speedup vs baseline: 3.1117x; 1.0032x over previous
import functools

import jax
import jax.numpy as jnp
from jax import lax
from jax.experimental import pallas as pl
from jax.experimental.pallas import tpu as pltpu
from jax.experimental.pallas import tpu_sc as plsc

N = 10000
E = 160000
C = 128
H = 8
DH = 16
CUTOFF = 5.0

NC = 2
NS = 16
NW = NC * NS

GB1 = 200
GB2 = 40
G_PER_W = E // NW
SB = 80
S_PER_T = E // NS
ROWS_PER_T = 624
ROWS_REM = N - NS * ROWS_PER_T
ZROWS = 48

BN = 400
BE = 400

_f32 = jnp.float32


def _silu(x):
    return x * jax.nn.sigmoid(x)


def _node_dense_body(s_ref, v_ref, ln_g, ln_b, wq, bq, wk, bk, wv1, bv1,
                     wv2, bv2, wvq, wvk,
                     q_ref, k_ref, val_ref, vq_ref, vdot_ref):
    s = s_ref[...]
    mu = jnp.mean(s, axis=-1, keepdims=True)
    xc = s - mu
    var = jnp.mean(xc * xc, axis=-1, keepdims=True)
    x = xc * lax.rsqrt(var + 1e-5) * ln_g[...] + ln_b[...]
    q_ref[...] = jnp.dot(x, wq[...], preferred_element_type=_f32) + bq[...]
    k_ref[...] = jnp.dot(x, wk[...], preferred_element_type=_f32) + bk[...]
    hv = _silu(jnp.dot(x, wv1[...], preferred_element_type=_f32) + bv1[...])
    val_ref[...] = jnp.dot(hv, wv2[...], preferred_element_type=_f32) + bv2[...]
    v2 = v_ref[...]
    vdot = jnp.zeros_like(s)
    for i in range(3):
        sl = slice(i * C, (i + 1) * C)
        vq_i = jnp.dot(v2[:, sl], wvq[...], preferred_element_type=_f32)
        vk_i = jnp.dot(v2[:, sl], wvk[...], preferred_element_type=_f32)
        vq_ref[:, sl] = vq_i
        vdot = vdot + vq_i * vk_i
    vdot_ref[...] = vdot


def _node_dense(s, v2d, p):
    grid = (N // BN,)
    row = lambda i: (i, 0)
    cst = lambda i: (0, 0)
    out = pl.pallas_call(
        _node_dense_body,
        grid=grid,
        in_specs=[
            pl.BlockSpec((BN, C), row), pl.BlockSpec((BN, 3 * C), row),
            pl.BlockSpec((1, C), cst), pl.BlockSpec((1, C), cst),
            pl.BlockSpec((C, C), cst), pl.BlockSpec((1, C), cst),
            pl.BlockSpec((C, C), cst), pl.BlockSpec((1, C), cst),
            pl.BlockSpec((C, C), cst), pl.BlockSpec((1, C), cst),
            pl.BlockSpec((C, 3 * C), cst), pl.BlockSpec((1, 3 * C), cst),
            pl.BlockSpec((C, C), cst), pl.BlockSpec((C, C), cst),
        ],
        out_specs=[
            pl.BlockSpec((BN, C), row), pl.BlockSpec((BN, C), row),
            pl.BlockSpec((BN, 3 * C), row), pl.BlockSpec((BN, 3 * C), row),
            pl.BlockSpec((BN, C), row),
        ],
        out_shape=[
            jax.ShapeDtypeStruct((N, C), _f32),
            jax.ShapeDtypeStruct((N, C), _f32),
            jax.ShapeDtypeStruct((N, 3 * C), _f32),
            jax.ShapeDtypeStruct((N, 3 * C), _f32),
            jax.ShapeDtypeStruct((N, C), _f32),
        ],
    )(s, v2d,
      p['ln_g'].reshape(1, C), p['ln_b'].reshape(1, C),
      p['Wq'], p['bq'].reshape(1, C), p['Wk'], p['bk'].reshape(1, C),
      p['Wv1'], p['bv1'].reshape(1, C), p['Wv2'], p['bv2'].reshape(1, 3 * C),
      p['Wvq'], p['Wvk'])
    return out


def _sc_gather(q, k, val, v2d, src, dst):
    mesh = plsc.VectorSubcoreMesh(core_axis_name="c", subcore_axis_name="s")

    @functools.partial(
        pl.kernel,
        mesh=mesh,
        out_type=[
            jax.ShapeDtypeStruct((E, C), _f32),
            jax.ShapeDtypeStruct((E, C), _f32),
            jax.ShapeDtypeStruct((E, 3 * C), _f32),
            jax.ShapeDtypeStruct((E, 3 * C), _f32),
        ],
        scratch_types=[
            pltpu.VMEM((GB1,), jnp.int32),
            pltpu.VMEM((GB1,), jnp.int32),
            pltpu.VMEM((GB2,), jnp.int32),
            pltpu.VMEM((GB2,), jnp.int32),
            pltpu.VMEM((GB1, C), _f32),
            pltpu.VMEM((GB1, C), _f32),
            pltpu.VMEM((GB2, 3 * C), _f32),
            pltpu.VMEM((GB2, 3 * C), _f32),
            pltpu.SemaphoreType.DMA,
            pltpu.SemaphoreType.DMA,
            pltpu.SemaphoreType.DMA,
            pltpu.SemaphoreType.DMA,
        ],
    )
    def gather_kernel(q_hbm, k_hbm, val_hbm, v_hbm, src_hbm, dst_hbm,
                      qd_hbm, ks_hbm, vals_hbm, vs_hbm,
                      ia0, ia1, ib0, ib1, a0, a1, b0, b1, sg0, sg1, ss0, ss1):
        wid = lax.axis_index("s") * NC + lax.axis_index("c")
        base = wid * G_PER_W

        def run_phase(tbl, out, idx_hbm, gb, ix0, ix1, buf0, buf1):
            nb = G_PER_W // gb

            def ldix(j, ix):
                pltpu.sync_copy(idx_hbm.at[pl.ds(base + j * gb, gb)], ix)

            def gat(ix, buf, sem):
                pltpu.async_copy(tbl.at[ix], buf, sem)

            def wait_g(ix, buf, sem):
                pltpu.make_async_copy(tbl.at[ix], buf, sem).wait()

            def sto(j, buf, sem):
                pltpu.async_copy(buf, out.at[pl.ds(base + j * gb, gb)], sem)

            def wait_s(j, buf, sem):
                pltpu.make_async_copy(buf, out.at[pl.ds(base + j * gb, gb)], sem).wait()

            ldix(0, ix0)
            gat(ix0, buf0, sg0)

            @pl.loop(0, (nb + 1) // 2)
            def _(t):
                j0 = 2 * t
                j1 = j0 + 1
                wait_g(ix0, buf0, sg0)
                sto(j0, buf0, ss0)

                @pl.when(j1 < nb)
                def _():
                    ldix(j1, ix1)
                    gat(ix1, buf1, sg1)
                wait_s(j0, buf0, ss0)

                @pl.when(j0 + 2 < nb)
                def _():
                    ldix(j0 + 2, ix0)
                    gat(ix0, buf0, sg0)

                @pl.when(j1 < nb)
                def _():
                    wait_g(ix1, buf1, sg1)
                    sto(j1, buf1, ss1)
                    wait_s(j1, buf1, ss1)

        run_phase(q_hbm, qd_hbm, dst_hbm, GB1, ia0, ia1, a0, a1)
        run_phase(k_hbm, ks_hbm, src_hbm, GB1, ia0, ia1, a0, a1)
        run_phase(val_hbm, vals_hbm, src_hbm, GB2, ib0, ib1, b0, b1)
        run_phase(v_hbm, vs_hbm, src_hbm, GB2, ib0, ib1, b0, b1)

    return gather_kernel(q, k, val, v2d, src, dst)


def _edge_body(r_ref, qd_ref, ks_ref, vals_ref, vs_ref, dir_ref, d_ref,
               wsig, bsig, wra, bra,
               m0_ref, m1_ref, m2_ref, m3_ref):
    r = r_ref[...]
    rk = _silu(jnp.dot(r, wsig[...], preferred_element_type=_f32) + bsig[...])
    ra = jnp.dot(r, wra[...], preferred_element_type=_f32) + bra[...]
    prod = qd_ref[...] * ks_ref[...] * rk
    attn = jnp.sum(prod.reshape(BE, H, DH), axis=-1)
    d = d_ref[...]
    cc = 0.5 * (jnp.cos(d * (jnp.pi / CUTOFF)) + 1.0) * (d < CUTOFF).astype(_f32)
    attn = _silu(attn) * cc
    attn128 = jnp.broadcast_to(attn[:, :, None], (BE, H, DH)).reshape(BE, C)
    val_j = vals_ref[...] * ra
    m0_ref[...] = val_j[:, :C] * attn128
    vec1 = val_j[:, C:2 * C]
    vec2 = val_j[:, 2 * C:]
    vs = vs_ref[...]
    dirs = dir_ref[...]
    m1_ref[...] = vs[:, :C] * vec1 + dirs[:, 0:1] * vec2
    m2_ref[...] = vs[:, C:2 * C] * vec1 + dirs[:, 1:2] * vec2
    m3_ref[...] = vs[:, 2 * C:] * vec1 + dirs[:, 2:3] * vec2


def _edge_stage(r_ij, qd, ks, vals, vs, dir_ij, d2, p):
    grid = (E // BE,)
    row = lambda i: (i, 0)
    cst = lambda i: (0, 0)
    return pl.pallas_call(
        _edge_body,
        grid=grid,
        in_specs=[
            pl.BlockSpec((BE, C), row), pl.BlockSpec((BE, C), row),
            pl.BlockSpec((BE, C), row), pl.BlockSpec((BE, 3 * C), row),
            pl.BlockSpec((BE, 3 * C), row), pl.BlockSpec((BE, 3), row),
            pl.BlockSpec((BE, 1), row),
            pl.BlockSpec((C, C), cst), pl.BlockSpec((1, C), cst),
            pl.BlockSpec((C, 3 * C), cst), pl.BlockSpec((1, 3 * C), cst),
        ],
        out_specs=[pl.BlockSpec((BE, C), row)] * 4,
        out_shape=[jax.ShapeDtypeStruct((E, C), _f32)] * 4,
    )(r_ij, qd, ks, vals, vs, dir_ij, d2,
      p['Wsig'], p['bsig'].reshape(1, C), p['Wra'], p['bra'].reshape(1, 3 * C))


def _sc_scatter(m0, m1, m2, m3, dst):
    mesh = plsc.VectorSubcoreMesh(core_axis_name="c", subcore_axis_name="s")
    nb = S_PER_T // SB

    @functools.partial(
        pl.kernel,
        mesh=mesh,
        out_type=[jax.ShapeDtypeStruct((N, C), _f32)] * 4,
        scratch_types=[
            pltpu.VMEM((SB,), jnp.int32),
            pltpu.VMEM((SB,), jnp.int32),
            pltpu.VMEM((SB, C), _f32),
            pltpu.VMEM((SB, C), _f32),
            pltpu.VMEM((ZROWS, C), _f32),
            pltpu.VMEM_SHARED((N, C), _f32),
            pltpu.SemaphoreType.DMA,
            pltpu.SemaphoreType.DMA,
            pltpu.SemaphoreType.DMA,
            pltpu.SemaphoreType.DMA,
        ],
    )
    def scatter_kernel(m0_hbm, m1_hbm, m2_hbm, m3_hbm, dst_hbm,
                       ds_hbm, dv0_hbm, dv1_hbm, dv2_hbm,
                       ix0, ix1, mb0, mb1, zbuf, acc, sl0, sl1, sa0, sa1):
        c = lax.axis_index("c")
        s = lax.axis_index("s")

        @pl.loop(0, ZROWS)
        def _(i):
            @pl.loop(0, C, step=16)
            def _(j):
                zbuf[i, pl.ds(j, 16)] = jnp.zeros((16,), _f32)

        def process(m_hbm, out_hbm):
            @pl.loop(0, ROWS_PER_T // ZROWS)
            def _(t):
                pltpu.sync_copy(zbuf, acc.at[pl.ds(s * ROWS_PER_T + t * ZROWS, ZROWS)])

            @pl.when(s == 0)
            def _():
                pltpu.sync_copy(zbuf.at[pl.ds(0, ROWS_REM)],
                                acc.at[pl.ds(NS * ROWS_PER_T, ROWS_REM)])
            plsc.subcore_barrier()

            def load(j, buf, sem):
                pltpu.async_copy(m_hbm.at[pl.ds(s * S_PER_T + j * SB, SB)], buf, sem)

            def wait_l(j, buf, sem):
                pltpu.make_async_copy(m_hbm.at[pl.ds(s * S_PER_T + j * SB, SB)],
                                      buf, sem).wait()

            load(0, mb0, sl0)

            @pl.loop(0, nb // 2)
            def _(t):
                j0 = 2 * t
                j1 = j0 + 1
                pltpu.sync_copy(dst_hbm.at[pl.ds(s * S_PER_T + j0 * SB, SB)], ix0)
                wait_l(j0, mb0, sl0)
                load(j1, mb1, sl1)
                pltpu.sync_copy(mb0, acc.at[ix0], add=True)
                pltpu.sync_copy(dst_hbm.at[pl.ds(s * S_PER_T + j1 * SB, SB)], ix1)
                wait_l(j1, mb1, sl1)

                @pl.when(j0 + 2 < nb)
                def _():
                    load(j0 + 2, mb0, sl0)
                pltpu.sync_copy(mb1, acc.at[ix1], add=True)

            if nb % 2 == 1:
                jt = nb - 1
                pltpu.sync_copy(dst_hbm.at[pl.ds(s * S_PER_T + jt * SB, SB)], ix0)
                wait_l(jt, mb0, sl0)
                pltpu.sync_copy(mb0, acc.at[ix0], add=True)
            plsc.subcore_barrier()
            pltpu.sync_copy(acc.at[pl.ds(s * ROWS_PER_T, ROWS_PER_T)],
                            out_hbm.at[pl.ds(s * ROWS_PER_T, ROWS_PER_T)])

            @pl.when(s == 0)
            def _():
                pltpu.sync_copy(acc.at[pl.ds(NS * ROWS_PER_T, ROWS_REM)],
                                out_hbm.at[pl.ds(NS * ROWS_PER_T, ROWS_REM)])
            plsc.subcore_barrier()

        @pl.when(c == 0)
        def _():
            process(m0_hbm, ds_hbm)
            process(m1_hbm, dv0_hbm)

        @pl.when(c == 1)
        def _():
            process(m2_hbm, dv1_hbm)
            process(m3_hbm, dv2_hbm)

    return scatter_kernel(m0, m1, m2, m3, dst)


def _post_body(ds_ref, dv0_ref, dv1_ref, dv2_ref, s_ref, v_ref, vq_ref,
               vdot_ref, ws1, bs1, ws2, bs2, sout_ref, vout_ref):
    hs = _silu(jnp.dot(ds_ref[...], ws1[...], preferred_element_type=_f32) + bs1[...])
    o = jnp.dot(hs, ws2[...], preferred_element_type=_f32) + bs2[...]
    o1 = o[:, :C]
    o2 = o[:, C:2 * C]
    o3 = o[:, 2 * C:]
    sout_ref[...] = s_ref[...] + o2 + o3 * vdot_ref[...]
    v2 = v_ref[...]
    vq = vq_ref[...]
    dvs = (dv0_ref, dv1_ref, dv2_ref)
    for i in range(3):
        sl = slice(i * C, (i + 1) * C)
        vout_ref[:, i, :] = v2[:, sl] + dvs[i][...] + o1 * vq[:, sl]


def _post_stage(ds, dv0, dv1, dv2, s, v2d, vq2d, vdot, p):
    grid = (N // BN,)
    row = lambda i: (i, 0)
    cst = lambda i: (0, 0)
    return pl.pallas_call(
        _post_body,
        grid=grid,
        in_specs=[
            pl.BlockSpec((BN, C), row), pl.BlockSpec((BN, C), row),
            pl.BlockSpec((BN, C), row), pl.BlockSpec((BN, C), row),
            pl.BlockSpec((BN, C), row), pl.BlockSpec((BN, 3 * C), row),
            pl.BlockSpec((BN, 3 * C), row), pl.BlockSpec((BN, C), row),
            pl.BlockSpec((C, C), cst), pl.BlockSpec((1, C), cst),
            pl.BlockSpec((C, 3 * C), cst), pl.BlockSpec((1, 3 * C), cst),
        ],
        out_specs=[
            pl.BlockSpec((BN, C), row),
            pl.BlockSpec((BN, 3, C), lambda i: (i, 0, 0)),
        ],
        out_shape=[
            jax.ShapeDtypeStruct((N, C), _f32),
            jax.ShapeDtypeStruct((N, 3, C), _f32),
        ],
    )(ds, dv0, dv1, dv2, s, v2d, vq2d, vdot,
      p['Ws1'], p['bs1'].reshape(1, C), p['Ws2'], p['bs2'].reshape(1, 3 * C))


@jax.jit
def _impl(edge_index, s, v, dir_ij, r_ij, d_ij, params):
    src = edge_index[0]
    dst = edge_index[1]
    v2d = v.reshape(N, 3 * C)
    d2 = d_ij.reshape(E, 1)

    q, k, val, vq2d, vdot = _node_dense(s, v2d, params)
    qd, ks, vals, vs = _sc_gather(q, k, val, v2d, src, dst)
    m0, m1, m2, m3 = _edge_stage(r_ij, qd, ks, vals, vs, dir_ij, d2, params)
    ds, dv0, dv1, dv2 = _sc_scatter(m0, m1, m2, m3, dst)
    s_out, v_out = _post_stage(ds, dv0, dv1, dv2, s, v2d, vq2d, vdot, params)
    return (s_out, v_out)


def kernel(edge_index, s, v, dir_ij, r_ij, d_ij, params):
    return _impl(edge_index, s, v, dir_ij, r_ij, d_ij, params)


# speedup vs baseline: 4.2144x; 1.3544x over previous
import functools

import jax
import jax.numpy as jnp
from jax import lax
from jax.experimental import pallas as pl
from jax.experimental.pallas import tpu as pltpu
from jax.experimental.pallas import tpu_sc as plsc

N = 10000
E = 160000
C = 128
H = 8
DH = 16
CUTOFF = 5.0

NC = 2
NS = 16
NW = NC * NS

GB1 = 200
GB2 = 40
G_PER_W = E // NW
SB = 80
S_PER_T = E // NS
ROWS_PER_T = 624
ROWS_REM = N - NS * ROWS_PER_T
ZROWS = 48

BN = 400
BE = 400

_f32 = jnp.float32


def _silu(x):
    return x * jax.nn.sigmoid(x)


def _node_dense_body(s_ref, v_ref, ln_g, ln_b, wq, bq, wk, bk, wv1, bv1,
                     wv2, bv2, wvq, wvk,
                     q_ref, k_ref, val_ref, vq_ref, vdot_ref):
    s = s_ref[...]
    mu = jnp.mean(s, axis=-1, keepdims=True)
    xc = s - mu
    var = jnp.mean(xc * xc, axis=-1, keepdims=True)
    x = xc * lax.rsqrt(var + 1e-5) * ln_g[...] + ln_b[...]
    q_ref[...] = jnp.dot(x, wq[...], preferred_element_type=_f32) + bq[...]
    k_ref[...] = jnp.dot(x, wk[...], preferred_element_type=_f32) + bk[...]
    hv = _silu(jnp.dot(x, wv1[...], preferred_element_type=_f32) + bv1[...])
    val_ref[...] = jnp.dot(hv, wv2[...], preferred_element_type=_f32) + bv2[...]
    v2 = v_ref[...]
    vdot = jnp.zeros_like(s)
    for i in range(3):
        sl = slice(i * C, (i + 1) * C)
        vq_i = jnp.dot(v2[:, sl], wvq[...], preferred_element_type=_f32)
        vk_i = jnp.dot(v2[:, sl], wvk[...], preferred_element_type=_f32)
        vq_ref[:, sl] = vq_i
        vdot = vdot + vq_i * vk_i
    vdot_ref[...] = vdot


def _node_dense(s, v2d, p):
    grid = (N // BN,)
    row = lambda i: (i, 0)
    cst = lambda i: (0, 0)
    out = pl.pallas_call(
        _node_dense_body,
        grid=grid,
        in_specs=[
            pl.BlockSpec((BN, C), row), pl.BlockSpec((BN, 3 * C), row),
            pl.BlockSpec((1, C), cst), pl.BlockSpec((1, C), cst),
            pl.BlockSpec((C, C), cst), pl.BlockSpec((1, C), cst),
            pl.BlockSpec((C, C), cst), pl.BlockSpec((1, C), cst),
            pl.BlockSpec((C, C), cst), pl.BlockSpec((1, C), cst),
            pl.BlockSpec((C, 3 * C), cst), pl.BlockSpec((1, 3 * C), cst),
            pl.BlockSpec((C, C), cst), pl.BlockSpec((C, C), cst),
        ],
        out_specs=[
            pl.BlockSpec((BN, C), row), pl.BlockSpec((BN, C), row),
            pl.BlockSpec((BN, 3 * C), row), pl.BlockSpec((BN, 3 * C), row),
            pl.BlockSpec((BN, C), row),
        ],
        out_shape=[
            jax.ShapeDtypeStruct((N, C), _f32),
            jax.ShapeDtypeStruct((N, C), _f32),
            jax.ShapeDtypeStruct((N, 3 * C), _f32),
            jax.ShapeDtypeStruct((N, 3 * C), _f32),
            jax.ShapeDtypeStruct((N, C), _f32),
        ],
    )(s, v2d,
      p['ln_g'].reshape(1, C), p['ln_b'].reshape(1, C),
      p['Wq'], p['bq'].reshape(1, C), p['Wk'], p['bk'].reshape(1, C),
      p['Wv1'], p['bv1'].reshape(1, C), p['Wv2'], p['bv2'].reshape(1, 3 * C),
      p['Wvq'], p['Wvk'])
    return out


def _sc_gather(q, k, val, v2d, src, dst):
    mesh = plsc.VectorSubcoreMesh(core_axis_name="c", subcore_axis_name="s")

    @functools.partial(
        pl.kernel,
        mesh=mesh,
        out_type=[
            jax.ShapeDtypeStruct((E, C), _f32),
            jax.ShapeDtypeStruct((E, C), _f32),
            jax.ShapeDtypeStruct((E, 3 * C), _f32),
            jax.ShapeDtypeStruct((E, 3 * C), _f32),
        ],
        scratch_types=[
            pltpu.VMEM((GB1,), jnp.int32),
            pltpu.VMEM((GB1,), jnp.int32),
            pltpu.VMEM((GB2,), jnp.int32),
            pltpu.VMEM((GB2,), jnp.int32),
            pltpu.VMEM((GB1, C), _f32),
            pltpu.VMEM((GB1, C), _f32),
            pltpu.VMEM((GB2, 3 * C), _f32),
            pltpu.VMEM((GB2, 3 * C), _f32),
            pltpu.SemaphoreType.DMA,
            pltpu.SemaphoreType.DMA,
            pltpu.SemaphoreType.DMA,
            pltpu.SemaphoreType.DMA,
        ],
    )
    def gather_kernel(q_hbm, k_hbm, val_hbm, v_hbm, src_hbm, dst_hbm,
                      qd_hbm, ks_hbm, vals_hbm, vs_hbm,
                      ia0, ia1, ib0, ib1, a0, a1, b0, b1, sg0, sg1, ss0, ss1):
        wid = lax.axis_index("s") * NC + lax.axis_index("c")
        base = wid * G_PER_W

        def run_phase(tbl, out, idx_hbm, gb, ix0, ix1, buf0, buf1):
            nb = G_PER_W // gb

            def ldix(j, ix):
                pltpu.sync_copy(idx_hbm.at[pl.ds(base + j * gb, gb)], ix)

            def gat(ix, buf, sem):
                pltpu.async_copy(tbl.at[ix], buf, sem)

            def wait_g(ix, buf, sem):
                pltpu.make_async_copy(tbl.at[ix], buf, sem).wait()

            def sto(j, buf, sem):
                pltpu.async_copy(buf, out.at[pl.ds(base + j * gb, gb)], sem)

            def wait_s(j, buf, sem):
                pltpu.make_async_copy(buf, out.at[pl.ds(base + j * gb, gb)], sem).wait()

            ldix(0, ix0)
            gat(ix0, buf0, sg0)

            @pl.loop(0, (nb + 1) // 2)
            def _(t):
                j0 = 2 * t
                j1 = j0 + 1
                wait_g(ix0, buf0, sg0)
                sto(j0, buf0, ss0)

                @pl.when(j1 < nb)
                def _():
                    ldix(j1, ix1)
                    gat(ix1, buf1, sg1)
                wait_s(j0, buf0, ss0)

                @pl.when(j0 + 2 < nb)
                def _():
                    ldix(j0 + 2, ix0)
                    gat(ix0, buf0, sg0)

                @pl.when(j1 < nb)
                def _():
                    wait_g(ix1, buf1, sg1)
                    sto(j1, buf1, ss1)
                    wait_s(j1, buf1, ss1)

        run_phase(q_hbm, qd_hbm, dst_hbm, GB1, ia0, ia1, a0, a1)
        run_phase(k_hbm, ks_hbm, src_hbm, GB1, ia0, ia1, a0, a1)
        run_phase(val_hbm, vals_hbm, src_hbm, GB2, ib0, ib1, b0, b1)
        run_phase(v_hbm, vs_hbm, src_hbm, GB2, ib0, ib1, b0, b1)

    return gather_kernel(q, k, val, v2d, src, dst)


def _edge_body(r_ref, qd_ref, ks_ref, vals_ref, vs_ref, dir_ref, d_ref,
               wsig, bsig, wra, bra, sel,
               m0_ref, m1_ref, m2_ref, m3_ref):
    r = r_ref[...]
    rk = _silu(jnp.dot(r, wsig[...], preferred_element_type=_f32) + bsig[...])
    ra = jnp.dot(r, wra[...], preferred_element_type=_f32) + bra[...]
    prod = qd_ref[...] * ks_ref[...] * rk
    attn_b = jnp.dot(prod, sel[...], preferred_element_type=_f32)
    d = d_ref[...]
    cc = 0.5 * (jnp.cos(d * (jnp.pi / CUTOFF)) + 1.0) * (d < CUTOFF).astype(_f32)
    attn128 = _silu(attn_b) * cc
    val_j = vals_ref[...] * ra
    m0_ref[...] = val_j[:, :C] * attn128
    vec1 = val_j[:, C:2 * C]
    vec2 = val_j[:, 2 * C:]
    vs = vs_ref[...]
    dirs = dir_ref[...]
    m1_ref[...] = vs[:, :C] * vec1 + dirs[:, 0:1] * vec2
    m2_ref[...] = vs[:, C:2 * C] * vec1 + dirs[:, 1:2] * vec2
    m3_ref[...] = vs[:, 2 * C:] * vec1 + dirs[:, 2:3] * vec2


def _edge_stage(r_ij, qd, ks, vals, vs, dir_ij, d2, p):
    grid = (E // BE,)
    row = lambda i: (i, 0)
    cst = lambda i: (0, 0)
    return pl.pallas_call(
        _edge_body,
        grid=grid,
        in_specs=[
            pl.BlockSpec((BE, C), row), pl.BlockSpec((BE, C), row),
            pl.BlockSpec((BE, C), row), pl.BlockSpec((BE, 3 * C), row),
            pl.BlockSpec((BE, 3 * C), row), pl.BlockSpec((BE, 3), row),
            pl.BlockSpec((BE, 1), row),
            pl.BlockSpec((C, C), cst), pl.BlockSpec((1, C), cst),
            pl.BlockSpec((C, 3 * C), cst), pl.BlockSpec((1, 3 * C), cst),
            pl.BlockSpec((C, C), cst),
        ],
        out_specs=[pl.BlockSpec((BE, C), row)] * 4,
        out_shape=[jax.ShapeDtypeStruct((E, C), _f32)] * 4,
    )(r_ij, qd, ks, vals, vs, dir_ij, d2,
      p['Wsig'], p['bsig'].reshape(1, C), p['Wra'], p['bra'].reshape(1, 3 * C),
      jnp.kron(jnp.eye(H, dtype=_f32), jnp.ones((DH, DH), _f32)))


def _sc_scatter(m0, m1, m2, m3, dst):
    mesh = plsc.VectorSubcoreMesh(core_axis_name="c", subcore_axis_name="s")
    nb = S_PER_T // SB

    @functools.partial(
        pl.kernel,
        mesh=mesh,
        out_type=[jax.ShapeDtypeStruct((N, C), _f32)] * 4,
        scratch_types=[
            pltpu.VMEM((SB,), jnp.int32),
            pltpu.VMEM((SB,), jnp.int32),
            pltpu.VMEM((SB, C), _f32),
            pltpu.VMEM((SB, C), _f32),
            pltpu.VMEM((ZROWS, C), _f32),
            pltpu.VMEM_SHARED((N, C), _f32),
            pltpu.SemaphoreType.DMA,
            pltpu.SemaphoreType.DMA,
            pltpu.SemaphoreType.DMA,
            pltpu.SemaphoreType.DMA,
        ],
    )
    def scatter_kernel(m0_hbm, m1_hbm, m2_hbm, m3_hbm, dst_hbm,
                       ds_hbm, dv0_hbm, dv1_hbm, dv2_hbm,
                       ix0, ix1, mb0, mb1, zbuf, acc, sl0, sl1, sa0, sa1):
        c = lax.axis_index("c")
        s = lax.axis_index("s")

        @pl.loop(0, ZROWS)
        def _(i):
            @pl.loop(0, C, step=16)
            def _(j):
                zbuf[i, pl.ds(j, 16)] = jnp.zeros((16,), _f32)

        def process(m_hbm, out_hbm):
            @pl.loop(0, ROWS_PER_T // ZROWS)
            def _(t):
                pltpu.sync_copy(zbuf, acc.at[pl.ds(s * ROWS_PER_T + t * ZROWS, ZROWS)])

            @pl.when(s == 0)
            def _():
                pltpu.sync_copy(zbuf.at[pl.ds(0, ROWS_REM)],
                                acc.at[pl.ds(NS * ROWS_PER_T, ROWS_REM)])
            plsc.subcore_barrier()

            def load(j, buf, sem):
                pltpu.async_copy(m_hbm.at[pl.ds(s * S_PER_T + j * SB, SB)], buf, sem)

            def wait_l(j, buf, sem):
                pltpu.make_async_copy(m_hbm.at[pl.ds(s * S_PER_T + j * SB, SB)],
                                      buf, sem).wait()

            load(0, mb0, sl0)

            @pl.loop(0, nb // 2)
            def _(t):
                j0 = 2 * t
                j1 = j0 + 1
                pltpu.sync_copy(dst_hbm.at[pl.ds(s * S_PER_T + j0 * SB, SB)], ix0)
                wait_l(j0, mb0, sl0)
                load(j1, mb1, sl1)
                pltpu.sync_copy(mb0, acc.at[ix0], add=True)
                pltpu.sync_copy(dst_hbm.at[pl.ds(s * S_PER_T + j1 * SB, SB)], ix1)
                wait_l(j1, mb1, sl1)

                @pl.when(j0 + 2 < nb)
                def _():
                    load(j0 + 2, mb0, sl0)
                pltpu.sync_copy(mb1, acc.at[ix1], add=True)

            if nb % 2 == 1:
                jt = nb - 1
                pltpu.sync_copy(dst_hbm.at[pl.ds(s * S_PER_T + jt * SB, SB)], ix0)
                wait_l(jt, mb0, sl0)
                pltpu.sync_copy(mb0, acc.at[ix0], add=True)
            plsc.subcore_barrier()
            pltpu.sync_copy(acc.at[pl.ds(s * ROWS_PER_T, ROWS_PER_T)],
                            out_hbm.at[pl.ds(s * ROWS_PER_T, ROWS_PER_T)])

            @pl.when(s == 0)
            def _():
                pltpu.sync_copy(acc.at[pl.ds(NS * ROWS_PER_T, ROWS_REM)],
                                out_hbm.at[pl.ds(NS * ROWS_PER_T, ROWS_REM)])
            plsc.subcore_barrier()

        @pl.when(c == 0)
        def _():
            process(m0_hbm, ds_hbm)
            process(m1_hbm, dv0_hbm)

        @pl.when(c == 1)
        def _():
            process(m2_hbm, dv1_hbm)
            process(m3_hbm, dv2_hbm)

    return scatter_kernel(m0, m1, m2, m3, dst)


def _post_body(ds_ref, dv0_ref, dv1_ref, dv2_ref, s_ref, v_ref, vq_ref,
               vdot_ref, ws1, bs1, ws2, bs2, sout_ref, vout_ref):
    hs = _silu(jnp.dot(ds_ref[...], ws1[...], preferred_element_type=_f32) + bs1[...])
    o = jnp.dot(hs, ws2[...], preferred_element_type=_f32) + bs2[...]
    o1 = o[:, :C]
    o2 = o[:, C:2 * C]
    o3 = o[:, 2 * C:]
    sout_ref[...] = s_ref[...] + o2 + o3 * vdot_ref[...]
    v2 = v_ref[...]
    vq = vq_ref[...]
    dvs = (dv0_ref, dv1_ref, dv2_ref)
    for i in range(3):
        sl = slice(i * C, (i + 1) * C)
        vout_ref[:, i, :] = v2[:, sl] + dvs[i][...] + o1 * vq[:, sl]


def _post_stage(ds, dv0, dv1, dv2, s, v2d, vq2d, vdot, p):
    grid = (N // BN,)
    row = lambda i: (i, 0)
    cst = lambda i: (0, 0)
    return pl.pallas_call(
        _post_body,
        grid=grid,
        in_specs=[
            pl.BlockSpec((BN, C), row), pl.BlockSpec((BN, C), row),
            pl.BlockSpec((BN, C), row), pl.BlockSpec((BN, C), row),
            pl.BlockSpec((BN, C), row), pl.BlockSpec((BN, 3 * C), row),
            pl.BlockSpec((BN, 3 * C), row), pl.BlockSpec((BN, C), row),
            pl.BlockSpec((C, C), cst), pl.BlockSpec((1, C), cst),
            pl.BlockSpec((C, 3 * C), cst), pl.BlockSpec((1, 3 * C), cst),
        ],
        out_specs=[
            pl.BlockSpec((BN, C), row),
            pl.BlockSpec((BN, 3, C), lambda i: (i, 0, 0)),
        ],
        out_shape=[
            jax.ShapeDtypeStruct((N, C), _f32),
            jax.ShapeDtypeStruct((N, 3, C), _f32),
        ],
    )(ds, dv0, dv1, dv2, s, v2d, vq2d, vdot,
      p['Ws1'], p['bs1'].reshape(1, C), p['Ws2'], p['bs2'].reshape(1, 3 * C))


@jax.jit
def _impl(edge_index, s, v, dir_ij, r_ij, d_ij, params):
    src = edge_index[0]
    dst = edge_index[1]
    v2d = v.reshape(N, 3 * C)
    d2 = d_ij.reshape(E, 1)

    q, k, val, vq2d, vdot = _node_dense(s, v2d, params)
    qd, ks, vals, vs = _sc_gather(q, k, val, v2d, src, dst)
    m0, m1, m2, m3 = _edge_stage(r_ij, qd, ks, vals, vs, dir_ij, d2, params)
    ds, dv0, dv1, dv2 = _sc_scatter(m0, m1, m2, m3, dst)
    s_out, v_out = _post_stage(ds, dv0, dv1, dv2, s, v2d, vq2d, vdot, params)
    return (s_out, v_out)


def kernel(edge_index, s, v, dir_ij, r_ij, d_ij, params):
    return _impl(edge_index, s, v, dir_ij, r_ij, d_ij, params)
